# baseline (speedup 1.0000x reference)
import functools

import jax
import jax.numpy as jnp
from jax import lax
from jax.experimental import pallas as pl
from jax.experimental.pallas import tpu as pltpu
from jax.experimental.pallas import tpu_sc as plsc

_T = 8
_B = 16384
_L = 20
_N = _B * _L
_TOT = _T * _N
_OFF_TOT = _T * _B + 1
_HAS_W = (True, False, True, False, True, False, True, False)
_W_TABLES = (0, 2, 4, 6)
_ONES_TABLES = (1, 3, 5, 7)

_NC = 2
_NS = 16
_NW = _NC * _NS
_WCH = _N // _NW
_OFF_CH = (_T * _B) // _NW
_WPT = _B // _OFF_CH

_LANES = 16


def _sc_body(
    o0, o1, o2, o3, o4, o5, o6, o7,
    w0, w2, w4, w6,
    out_off, out_w,
    wb0, wb1, wb2, wb3,
    ones_v, offbuf_v,
    sem_gw, sem_sw, sem_off,
):
    off_in = (o0, o1, o2, o3, o4, o5, o6, o7)
    w_in = (w0, w2, w4, w6)
    wbufs = (wb0, wb1, wb2, wb3)

    c = lax.axis_index("c")
    s = lax.axis_index("s")
    wid = s * _NC + c
    base = wid * _WCH

    wg = []
    for k in range(4):
        h = pltpu.make_async_copy(
            w_in[k].at[pl.ds(base, _WCH)], wbufs[k], sem_gw
        )
        h.start()
        wg.append(h)

    part_start = (wid % _WPT) * _OFF_CH
    for t in range(_T):
        @pl.when(wid // _WPT == t)
        def _(t=t):
            pltpu.make_async_copy(
                off_in[t].at[pl.ds(part_start, _OFF_CH)],
                offbuf_v.at[pl.ds(0, _OFF_CH)],
                sem_off,
            ).start()

    ones_vec = jnp.full((_LANES,), 1.0, dtype=jnp.float32)

    def fill_body(i, carry):
        ones_v[pl.ds(i * _LANES, _LANES)] = ones_vec
        return carry

    lax.fori_loop(0, _WCH // _LANES, fill_body, 0)

    scat = []
    for t in _ONES_TABLES:
        h = pltpu.make_async_copy(
            ones_v, out_w.at[pl.ds(t * _N + base, _WCH)], sem_sw
        )
        h.start()
        scat.append(h)

    for k, t in enumerate(_W_TABLES):
        wg[k].wait()
        h = pltpu.make_async_copy(
            wbufs[k], out_w.at[pl.ds(t * _N + base, _WCH)], sem_sw
        )
        h.start()
        scat.append(h)

    pltpu.make_async_copy(
        off_in[0].at[pl.ds(0, _OFF_CH)],
        offbuf_v.at[pl.ds(0, _OFF_CH)],
        sem_off,
    ).wait()

    addend = jnp.broadcast_to((wid // _WPT) * _N, (_LANES,)).astype(jnp.int32)

    def add_body(i, carry):
        sl = pl.ds(i * _LANES, _LANES)
        offbuf_v[sl] = offbuf_v[sl] + addend
        return carry

    lax.fori_loop(0, _OFF_CH // _LANES, add_body, 0)

    @pl.when(wid == _NW - 1)
    def _():
        offbuf_v[pl.ds(_OFF_CH, _LANES)] = jnp.full(
            (_LANES,), _TOT, dtype=jnp.int32
        )
        pltpu.sync_copy(
            offbuf_v.at[pl.ds(0, _OFF_CH + 1)],
            out_off.at[pl.ds(wid * _OFF_CH, _OFF_CH + 1)],
        )

    @pl.when(wid != _NW - 1)
    def _():
        pltpu.sync_copy(
            offbuf_v.at[pl.ds(0, _OFF_CH)],
            out_off.at[pl.ds(wid * _OFF_CH, _OFF_CH)],
        )

    for h in scat:
        h.wait()


_sc_prep = functools.partial(
    pl.kernel,
    mesh=plsc.VectorSubcoreMesh(core_axis_name="c", subcore_axis_name="s"),
    out_type=[
        jax.ShapeDtypeStruct((_OFF_TOT,), jnp.int32),
        jax.ShapeDtypeStruct((_TOT,), jnp.float32),
    ],
    scratch_types=(
        [pltpu.VMEM((_WCH,), jnp.float32) for _ in range(4)]
        + [
            pltpu.VMEM((_WCH,), jnp.float32),
            pltpu.VMEM((_OFF_CH + _LANES,), jnp.int32),
            pltpu.SemaphoreType.DMA,
            pltpu.SemaphoreType.DMA,
            pltpu.SemaphoreType.DMA,
        ]
    ),
)(_sc_body)


_BC = 32768
_C = _N // _BC


def _tc_body(*refs):
    idx_refs = refs[:_T]
    out_ref = refs[_T]
    for t in range(_T):
        out_ref[t, :] = idx_refs[t][0, :]


_tc_concat = pl.pallas_call(
    _tc_body,
    grid=(_C,),
    in_specs=[
        pl.BlockSpec((1, _BC), lambda c: (0, c)) for _ in range(_T)
    ],
    out_specs=pl.BlockSpec((_T, _BC), lambda c: (0, c)),
    out_shape=jax.ShapeDtypeStruct((_T, _N), jnp.int32),
)


def kernel(
    indices_0, indices_1, indices_2, indices_3,
    indices_4, indices_5, indices_6, indices_7,
    offsets_0, offsets_1, offsets_2, offsets_3,
    offsets_4, offsets_5, offsets_6, offsets_7,
    weights_0, weights_1, weights_2, weights_3,
    weights_4, weights_5, weights_6, weights_7,
):
    combined_offsets, per_sample_weights = _sc_prep(
        offsets_0, offsets_1, offsets_2, offsets_3,
        offsets_4, offsets_5, offsets_6, offsets_7,
        weights_0, weights_2, weights_4, weights_6,
    )
    combined_indices = _tc_concat(
        indices_0.reshape(1, _N), indices_1.reshape(1, _N),
        indices_2.reshape(1, _N), indices_3.reshape(1, _N),
        indices_4.reshape(1, _N), indices_5.reshape(1, _N),
        indices_6.reshape(1, _N), indices_7.reshape(1, _N),
    ).reshape(_TOT)
    return combined_indices, combined_offsets, per_sample_weights
